# Initial kernel scaffold; baseline (speedup 1.0000x reference)
import jax
import jax.numpy as jnp
from jax.experimental import pallas as pl


def kernel(x, edge_index, W1a, b1a, W1b, b1b, g1, be1, W2a, b2a, W2b, b2b, g2, be2):
    raise NotImplementedError("write your pallas kernel here")


# speedup vs baseline: 5.6223x; 5.6223x over previous
import functools

import jax
import jax.numpy as jnp
from jax import lax
from jax.experimental import pallas as pl
from jax.experimental.pallas import tpu as pltpu
from jax.experimental.pallas import tpu_sc as plsc

N = 10000
E = 320000
D = 128

NC = 2
NS = 16
NW = NC * NS
EPW = E // NW
CHUNK = 128
NFULL = EPW // CHUNK
TAIL = EPW - NFULL * CHUNK
N_PAD = 10240
RPT = N_PAD // NS


def _seg_sum_body(x_hbm, src_hbm, dst_hbm, zeros_hbm, out_hbm,
                  src_v, dst_v, src_t, dst_t, rows_v, rows_t, acc_sh, sem):
    c = lax.axis_index("c")
    s = lax.axis_index("s")
    wid = c * NS + s
    r0 = s * RPT
    pltpu.sync_copy(zeros_hbm, acc_sh.at[pl.ds(r0, RPT)])
    plsc.subcore_barrier()

    base = wid * EPW

    def step(i, _):
        off = base + i * CHUNK
        pltpu.sync_copy(src_hbm.at[pl.ds(off, CHUNK)], src_v)
        pltpu.sync_copy(dst_hbm.at[pl.ds(off, CHUNK)], dst_v)
        pltpu.async_copy(x_hbm.at[src_v], rows_v, sem).wait()
        pltpu.sync_copy(rows_v, acc_sh.at[dst_v], add=True)
        return 0

    lax.fori_loop(0, NFULL, step, 0)

    if TAIL:
        off = base + NFULL * CHUNK
        pltpu.sync_copy(src_hbm.at[pl.ds(off, TAIL)], src_t)
        pltpu.sync_copy(dst_hbm.at[pl.ds(off, TAIL)], dst_t)
        pltpu.async_copy(x_hbm.at[src_t], rows_t, sem).wait()
        pltpu.sync_copy(rows_t, acc_sh.at[dst_t], add=True)

    plsc.subcore_barrier()
    pltpu.sync_copy(acc_sh.at[pl.ds(r0, RPT)], out_hbm.at[c, pl.ds(r0, RPT)])


_seg_sum = pl.kernel(
    _seg_sum_body,
    out_type=jax.ShapeDtypeStruct((NC, N_PAD, D), jnp.float32),
    mesh=plsc.VectorSubcoreMesh(core_axis_name="c", subcore_axis_name="s"),
    scratch_types=[
        pltpu.VMEM((CHUNK,), jnp.int32),
        pltpu.VMEM((CHUNK,), jnp.int32),
        pltpu.VMEM((max(TAIL, 8),), jnp.int32),
        pltpu.VMEM((max(TAIL, 8),), jnp.int32),
        pltpu.VMEM((CHUNK, D), jnp.float32),
        pltpu.VMEM((max(TAIL, 8), D), jnp.float32),
        pltpu.VMEM_SHARED((N_PAD, D), jnp.float32),
        pltpu.SemaphoreType.DMA,
    ],
)


def _dense_body(add_residual, h_ref, p_ref, wa_ref, ba_ref, wb_ref, bb_ref,
                g_ref, be_ref, res_ref, o_ref):
    h = h_ref[...] + p_ref[0] + p_ref[1]
    t = jnp.dot(h, wa_ref[...], preferred_element_type=jnp.float32) + ba_ref[...]
    t = jnp.maximum(t, 0.0)
    t = jnp.dot(t, wb_ref[...], preferred_element_type=jnp.float32) + bb_ref[...]
    m = jnp.mean(t, axis=0, keepdims=True)
    v = jnp.mean((t - m) ** 2, axis=0, keepdims=True)
    t = (t - m) / jnp.sqrt(v + 1e-5) * g_ref[...] + be_ref[...]
    t = jnp.maximum(t, 0.0)
    if add_residual:
        t = t + res_ref[...]
    o_ref[...] = t


def _dense(h, p, wa, ba, wb, bb, g, be, res, add_residual):
    body = functools.partial(_dense_body, add_residual)
    return pl.pallas_call(
        body,
        out_shape=jax.ShapeDtypeStruct((N, D), jnp.float32),
    )(h, p, wa, ba.reshape(1, D), wb, bb.reshape(1, D),
      g.reshape(1, D), be.reshape(1, D), res)


def kernel(x, edge_index, W1a, b1a, W1b, b1b, g1, be1, W2a, b2a, W2b, b2b,
           g2, be2):
    src = edge_index[0]
    dst = edge_index[1]
    zeros = jnp.zeros((RPT, D), jnp.float32)
    p1 = _seg_sum(x, src, dst, zeros)[:, :N, :]
    h = _dense(x, p1, W1a, b1a, W1b, b1b, g1, be1, x, add_residual=False)
    p2 = _seg_sum(h, src, dst, zeros)[:, :N, :]
    out = _dense(h, p2, W2a, b2a, W2b, b2b, g2, be2, x, add_residual=True)
    return out


# speedup vs baseline: 11.0333x; 1.9624x over previous
import functools

import jax
import jax.numpy as jnp
from jax import lax
from jax.experimental import pallas as pl
from jax.experimental.pallas import tpu as pltpu
from jax.experimental.pallas import tpu_sc as plsc

N = 10000
E = 320000
D = 128

NC = 2
NS = 16
NW = NC * NS
EPW = E // NW
CHUNK = 128
NCH = EPW // CHUNK
TAIL = EPW - NCH * CHUNK
N_PAD = 10240
RPT = N_PAD // NS
NIB = 4
IPF = 3


def _seg_sum_body(x_hbm, src_hbm, dst_hbm, zeros_hbm, out_hbm,
                  src_ring, dst_ring, src_t, dst_t, rb0, rb1,
                  acc_sh, tsem, is0, is1, is2, is3, gs0, gs1, ss0, ss1):
    rows = [rb0, rb1]
    gsem = [gs0, gs1]
    ssem = [ss0, ss1]
    isem = [is0, is1, is2, is3]
    ci = lax.axis_index("c")
    si = lax.axis_index("s")
    wid = ci * NS + si
    base = wid * EPW
    r0 = si * RPT

    def idx_start(c, ib):
        off = base + c * CHUNK
        pltpu.async_copy(src_hbm.at[pl.ds(off, CHUNK)], src_ring.at[ib],
                         isem[ib])
        pltpu.async_copy(dst_hbm.at[pl.ds(off, CHUNK)], dst_ring.at[ib],
                         isem[ib])

    def idx_wait(ib):
        pltpu.make_async_copy(src_hbm.at[pl.ds(base, CHUNK)],
                              src_ring.at[ib], isem[ib]).wait()
        pltpu.make_async_copy(dst_hbm.at[pl.ds(base, CHUNK)],
                              dst_ring.at[ib], isem[ib]).wait()

    def gather_start(b, ib):
        pltpu.async_copy(x_hbm.at[src_ring.at[ib]], rows[b], gsem[b])

    def gather_wait(b):
        pltpu.make_async_copy(x_hbm.at[src_ring.at[0]], rows[b],
                              gsem[b]).wait()

    def scatter_start(b, ib):
        pltpu.async_copy(rows[b], acc_sh.at[dst_ring.at[ib]], ssem[b],
                         add=True)

    def scatter_wait(b):
        pltpu.make_async_copy(rows[b], acc_sh.at[dst_ring.at[0]],
                              ssem[b]).wait()

    def step(c, k, swait=True, istart=True, gstart=True):
        b = k % 2
        pb = (b + 1) % 2
        if swait:
            scatter_wait(pb)
        if istart:
            idx_start(c + IPF, (k + IPF) % NIB)
        if gstart:
            idx_wait((k + 1) % NIB)
            gather_start(pb, (k + 1) % NIB)
        gather_wait(b)
        scatter_start(b, k)

    for c in range(IPF):
        idx_start(c, c)
    toff = base + NCH * CHUNK
    pltpu.async_copy(src_hbm.at[pl.ds(toff, TAIL)], src_t, tsem)
    pltpu.async_copy(dst_hbm.at[pl.ds(toff, TAIL)], dst_t, tsem)

    pltpu.sync_copy(zeros_hbm, acc_sh.at[pl.ds(r0, RPT)])

    idx_wait(0)
    gather_start(0, 0)

    plsc.subcore_barrier()

    step(0, 0, swait=False)
    for c in range(1, NIB):
        step(c, c)

    def group(g, _):
        c0 = NIB + NIB * g
        for k in range(NIB):
            step(c0 + k, k)
        return 0

    lax.fori_loop(0, (NCH - 6 - NIB) // NIB, group, 0)

    for c in range(NCH - 6, NCH - IPF):
        step(c, c % NIB)
    for c in range(NCH - IPF, NCH - 1):
        step(c, c % NIB, istart=False)
    step(NCH - 1, (NCH - 1) % NIB, istart=False, gstart=False)
    scatter_wait((NCH - 1) % 2)

    pltpu.make_async_copy(src_hbm.at[pl.ds(toff, TAIL)], src_t, tsem).wait()
    pltpu.make_async_copy(dst_hbm.at[pl.ds(toff, TAIL)], dst_t, tsem).wait()
    pltpu.async_copy(x_hbm.at[src_t], rb0.at[pl.ds(0, TAIL)], gs0).wait()
    pltpu.sync_copy(rb0.at[pl.ds(0, TAIL)], acc_sh.at[dst_t], add=True)

    plsc.subcore_barrier()
    pltpu.sync_copy(acc_sh.at[pl.ds(r0, RPT)], out_hbm.at[ci, pl.ds(r0, RPT)])


_seg_sum = pl.kernel(
    _seg_sum_body,
    out_type=jax.ShapeDtypeStruct((NC, N_PAD, D), jnp.float32),
    mesh=plsc.VectorSubcoreMesh(core_axis_name="c", subcore_axis_name="s"),
    scratch_types=[
        pltpu.VMEM((NIB, CHUNK), jnp.int32),
        pltpu.VMEM((NIB, CHUNK), jnp.int32),
        pltpu.VMEM((max(TAIL, 8),), jnp.int32),
        pltpu.VMEM((max(TAIL, 8),), jnp.int32),
        pltpu.VMEM((CHUNK, D), jnp.float32),
        pltpu.VMEM((CHUNK, D), jnp.float32),
        pltpu.VMEM_SHARED((N_PAD, D), jnp.float32),
        pltpu.SemaphoreType.DMA,
        pltpu.SemaphoreType.DMA,
        pltpu.SemaphoreType.DMA,
        pltpu.SemaphoreType.DMA,
        pltpu.SemaphoreType.DMA,
        pltpu.SemaphoreType.DMA,
        pltpu.SemaphoreType.DMA,
        pltpu.SemaphoreType.DMA,
        pltpu.SemaphoreType.DMA,
    ],
)


def _dense_body(add_residual, h_ref, p_ref, wa_ref, ba_ref, wb_ref, bb_ref,
                g_ref, be_ref, res_ref, o_ref):
    h = h_ref[...] + p_ref[0] + p_ref[1]
    t = jnp.dot(h, wa_ref[...], preferred_element_type=jnp.float32) + ba_ref[...]
    t = jnp.maximum(t, 0.0)
    t = jnp.dot(t, wb_ref[...], preferred_element_type=jnp.float32) + bb_ref[...]
    m = jnp.mean(t, axis=0, keepdims=True)
    v = jnp.mean((t - m) ** 2, axis=0, keepdims=True)
    t = (t - m) / jnp.sqrt(v + 1e-5) * g_ref[...] + be_ref[...]
    t = jnp.maximum(t, 0.0)
    if add_residual:
        t = t + res_ref[...]
    o_ref[...] = t


def _dense(h, p, wa, ba, wb, bb, g, be, res, add_residual):
    body = functools.partial(_dense_body, add_residual)
    return pl.pallas_call(
        body,
        out_shape=jax.ShapeDtypeStruct((N, D), jnp.float32),
    )(h, p, wa, ba.reshape(1, D), wb, bb.reshape(1, D),
      g.reshape(1, D), be.reshape(1, D), res)


def kernel(x, edge_index, W1a, b1a, W1b, b1b, g1, be1, W2a, b2a, W2b, b2b,
           g2, be2):
    src = edge_index[0]
    dst = edge_index[1]
    zeros = jnp.zeros((RPT, D), jnp.float32)
    p1 = _seg_sum(x, src, dst, zeros)[:, :N, :]
    h = _dense(x, p1, W1a, b1a, W1b, b1b, g1, be1, x, add_residual=False)
    p2 = _seg_sum(h, src, dst, zeros)[:, :N, :]
    out = _dense(h, p2, W2a, b2a, W2b, b2b, g2, be2, x, add_residual=True)
    return out


# speedup vs baseline: 12.0064x; 1.0882x over previous
import functools

import jax
import jax.numpy as jnp
from jax import lax
from jax.experimental import pallas as pl
from jax.experimental.pallas import tpu as pltpu
from jax.experimental.pallas import tpu_sc as plsc

N = 10000
E = 320000
D = 128

NC = 2
NS = 16
NW = NC * NS
EPW = E // NW
CHUNK = 80
NCH = EPW // CHUNK
N_PAD = 10240
RPT = N_PAD // NS
NB = 4
PF = 2
NIB = 8
IPF = 6


def _seg_sum_body(x_hbm, src_hbm, dst_hbm, zeros_hbm, out_hbm,
                  src_ring, dst_ring, rb0, rb1, rb2, rb3,
                  acc_sh,
                  is0, is1, is2, is3, is4, is5, is6, is7,
                  gs0, gs1, gs2, gs3, ss0, ss1, ss2, ss3):
    rows = [rb0, rb1, rb2, rb3]
    gsem = [gs0, gs1, gs2, gs3]
    ssem = [ss0, ss1, ss2, ss3]
    isem = [is0, is1, is2, is3, is4, is5, is6, is7]
    ci = lax.axis_index("c")
    si = lax.axis_index("s")
    wid = ci * NS + si
    base = wid * EPW
    r0 = si * RPT

    def idx_start(c, ib):
        off = base + c * CHUNK
        pltpu.async_copy(src_hbm.at[pl.ds(off, CHUNK)], src_ring.at[ib],
                         isem[ib])
        pltpu.async_copy(dst_hbm.at[pl.ds(off, CHUNK)], dst_ring.at[ib],
                         isem[ib])

    def idx_wait(ib):
        pltpu.make_async_copy(src_hbm.at[pl.ds(base, CHUNK)],
                              src_ring.at[ib], isem[ib]).wait()
        pltpu.make_async_copy(dst_hbm.at[pl.ds(base, CHUNK)],
                              dst_ring.at[ib], isem[ib]).wait()

    def gather_start(b, ib):
        pltpu.async_copy(x_hbm.at[src_ring.at[ib]], rows[b], gsem[b])

    def gather_wait(b):
        pltpu.make_async_copy(x_hbm.at[src_ring.at[0]], rows[b],
                              gsem[b]).wait()

    def scatter_start(b, ib):
        pltpu.async_copy(rows[b], acc_sh.at[dst_ring.at[ib]], ssem[b],
                         add=True)

    def scatter_wait(b):
        pltpu.make_async_copy(rows[b], acc_sh.at[dst_ring.at[0]],
                              ssem[b]).wait()

    def step(c, k, swait=True, istart=True, gstart=True):
        b = k % NB
        pb = (k + PF) % NB
        if swait:
            scatter_wait(pb)
        if istart:
            idx_start(c + IPF, (k + IPF) % NIB)
        if gstart:
            idx_wait((k + PF) % NIB)
            gather_start(pb, (k + PF) % NIB)
        gather_wait(b)
        scatter_start(b, k % NIB)

    for j in range(IPF):
        idx_start(j, j)

    pltpu.sync_copy(zeros_hbm, acc_sh.at[pl.ds(r0, RPT)])

    idx_wait(0)
    gather_start(0, 0)
    idx_wait(1)
    gather_start(1, 1)

    plsc.subcore_barrier()

    step(0, 0, swait=False)
    step(1, 1, swait=False)
    for c in range(2, NIB):
        step(c, c)

    def group(g, _):
        c0 = NIB + NIB * g
        for k in range(NIB):
            step(c0 + k, k)
        return 0

    NGRP = (NCH - IPF - 1 - NIB) // NIB
    lax.fori_loop(0, NGRP, group, 0)

    CTOP = NIB + NGRP * NIB
    for c in range(CTOP, NCH - IPF):
        step(c, c % NIB)
    for c in range(NCH - IPF, NCH - PF):
        step(c, c % NIB, istart=False)
    for c in range(NCH - PF, NCH):
        step(c, c % NIB, istart=False, gstart=False)
    scatter_wait((NCH - PF) % NB)
    scatter_wait((NCH - 1) % NB)

    plsc.subcore_barrier()
    pltpu.sync_copy(acc_sh.at[pl.ds(r0, RPT)], out_hbm.at[ci, pl.ds(r0, RPT)])


_seg_sum = pl.kernel(
    _seg_sum_body,
    out_type=jax.ShapeDtypeStruct((NC, N_PAD, D), jnp.float32),
    mesh=plsc.VectorSubcoreMesh(core_axis_name="c", subcore_axis_name="s"),
    scratch_types=[
        pltpu.VMEM((NIB, CHUNK), jnp.int32),
        pltpu.VMEM((NIB, CHUNK), jnp.int32),
        pltpu.VMEM((CHUNK, D), jnp.float32),
        pltpu.VMEM((CHUNK, D), jnp.float32),
        pltpu.VMEM((CHUNK, D), jnp.float32),
        pltpu.VMEM((CHUNK, D), jnp.float32),
        pltpu.VMEM_SHARED((N_PAD, D), jnp.float32),
    ] + [pltpu.SemaphoreType.DMA] * 16,
)


def _dense_body(add_residual, h_ref, p_ref, wa_ref, ba_ref, wb_ref, bb_ref,
                g_ref, be_ref, res_ref, o_ref):
    h = h_ref[...] + p_ref[0, :N, :] + p_ref[1, :N, :]
    t = jnp.dot(h, wa_ref[...], preferred_element_type=jnp.float32) + ba_ref[...]
    t = jnp.maximum(t, 0.0)
    t = jnp.dot(t, wb_ref[...], preferred_element_type=jnp.float32) + bb_ref[...]
    m = jnp.mean(t, axis=0, keepdims=True)
    v = jnp.mean((t - m) ** 2, axis=0, keepdims=True)
    t = (t - m) / jnp.sqrt(v + 1e-5) * g_ref[...] + be_ref[...]
    t = jnp.maximum(t, 0.0)
    if add_residual:
        t = t + res_ref[...]
    o_ref[...] = t


def _dense(h, p, wa, ba, wb, bb, g, be, res, add_residual):
    body = functools.partial(_dense_body, add_residual)
    return pl.pallas_call(
        body,
        out_shape=jax.ShapeDtypeStruct((N, D), jnp.float32),
    )(h, p, wa, ba.reshape(1, D), wb, bb.reshape(1, D),
      g.reshape(1, D), be.reshape(1, D), res)


def kernel(x, edge_index, W1a, b1a, W1b, b1b, g1, be1, W2a, b2a, W2b, b2b,
           g2, be2):
    src = edge_index[0]
    dst = edge_index[1]
    zeros = jnp.zeros((RPT, D), jnp.float32)
    p1 = _seg_sum(x, src, dst, zeros)
    h = _dense(x, p1, W1a, b1a, W1b, b1b, g1, be1, x, add_residual=False)
    p2 = _seg_sum(h, src, dst, zeros)
    out = _dense(h, p2, W2a, b2a, W2b, b2b, g2, be2, x, add_residual=True)
    return out
